# baseline (speedup 1.0000x reference)
import functools

import jax
import jax.numpy as jnp
from jax import lax
from jax.experimental import pallas as pl
from jax.experimental.pallas import tpu as pltpu
from jax.experimental.pallas import tpu_sc as plsc

N = 10000
E = 320000
F_IN = 128
H = 128
C = 64
HH = 64
WA = 80

NB = 10
RB = N // NB

NC = 2
NS = 16
NW = NC * NS
K = 128
NCH = 80
EPW = NCH * K
EPAD = NW * EPW - E
ND = N + 16
RPT = N // NS


def _ring_pass(table, zeros, out, acc, src_v, dst_v, buf0, buf1, sem0, sem1,
               c, s):
    rz = ND // NS
    pltpu.sync_copy(zeros.at[pl.ds(s * rz, rz)], acc.at[pl.ds(s * rz, rz)])
    plsc.subcore_barrier()

    pltpu.async_copy(table.at[src_v.at[0]], buf0, sem0)
    pltpu.async_copy(table.at[src_v.at[1]], buf1, sem1)

    def body(i, carry):
        g0 = 2 * i
        g1 = 2 * i + 1
        pltpu.make_async_copy(table.at[src_v.at[g0]], buf0, sem0).wait()
        pltpu.sync_copy(buf0, acc.at[dst_v.at[g0]], add=True)

        @pl.when(g0 + 2 < NCH)
        def _():
            pltpu.async_copy(table.at[src_v.at[g0 + 2]], buf0, sem0)

        pltpu.make_async_copy(table.at[src_v.at[g1]], buf1, sem1).wait()
        pltpu.sync_copy(buf1, acc.at[dst_v.at[g1]], add=True)

        @pl.when(g1 + 2 < NCH)
        def _():
            pltpu.async_copy(table.at[src_v.at[g1 + 2]], buf1, sem1)

        return carry

    lax.fori_loop(0, NCH // 2, body, 0)

    plsc.subcore_barrier()
    pltpu.sync_copy(acc.at[pl.ds(s * RPT, RPT)],
                    out.at[c, pl.ds(s * RPT, RPT)])


def _make_seg_sum(width):
    mesh = plsc.VectorSubcoreMesh(core_axis_name="c", subcore_axis_name="s")

    @functools.partial(
        pl.kernel,
        mesh=mesh,
        out_type=jax.ShapeDtypeStruct((NC, N, width), jnp.float32),
        compiler_params=pltpu.CompilerParams(use_tc_tiling_on_sc=False),
        scratch_types=[
            pltpu.VMEM((NCH, K), jnp.int32),
            pltpu.VMEM((NCH, K), jnp.int32),
            pltpu.VMEM((K, width), jnp.float32),
            pltpu.VMEM((K, width), jnp.float32),
            pltpu.VMEM_SHARED((ND, width), jnp.float32),
            pltpu.SemaphoreType.DMA,
            pltpu.SemaphoreType.DMA,
        ],
    )
    def seg(table, src3, dst3, zeros, out, src_v, dst_v, buf0, buf1,
            acc, sem0, sem1):
        c = lax.axis_index("c")
        s = lax.axis_index("s")
        wid = c * NS + s
        pltpu.sync_copy(src3.at[wid], src_v)
        pltpu.sync_copy(dst3.at[wid], dst_v)
        _ring_pass(table, zeros, out, acc, src_v, dst_v, buf0, buf1,
                   sem0, sem1, c, s)

    return seg


_seg_sum_80 = _make_seg_sum(WA)
_seg_sum_64 = _make_seg_sum(C)


def _prep_body(x_ref, wn_ref, ws_ref, b_ref, ta_ref, tb_ref, xs_ref):
    x = x_ref[...]
    y = jnp.dot(x, wn_ref[...], preferred_element_type=jnp.float32)
    ta_ref[:, :HH] = y[:, :HH]
    lane = lax.broadcasted_iota(jnp.int32, (RB, WA - HH), 1)
    ta_ref[:, HH:] = jnp.where(lane == 0, 1.0, 0.0)
    tb_ref[...] = y[:, HH:]
    xs_ref[...] = (
        jnp.dot(x, ws_ref[...], preferred_element_type=jnp.float32) + b_ref[...]
    )


def _c1_body(xs_ref, acca_ref, accb_ref, h_ref, dinv_ref, sum_ref, ssq_ref):
    i = pl.program_id(0)
    a = acca_ref[0] + acca_ref[1]
    b = accb_ref[0] + accb_ref[1]
    deg = a[:, HH:HH + 1]
    dinv = 1.0 / jnp.maximum(deg, 1.0)
    agg = jnp.concatenate([a[:, :HH], b], axis=1)
    h = xs_ref[...] + agg * dinv
    h_ref[...] = h
    dinv_ref[...] = jnp.broadcast_to(dinv, (RB, C))

    @pl.when(i == 0)
    def _():
        sum_ref[...] = jnp.zeros_like(sum_ref)
        ssq_ref[...] = jnp.zeros_like(ssq_ref)

    sum_ref[0:1, :] += jnp.sum(h, axis=0, keepdims=True)
    ssq_ref[0:1, :] += jnp.sum(h * h, axis=0, keepdims=True)


def _c2_body(h_ref, sum_ref, ssq_ref, g_ref, be_ref, ws2_ref, wn2_ref, b2_ref,
             y2_ref, s2b_ref):
    mu = sum_ref[0, :] * (1.0 / N)
    var = ssq_ref[0, :] * (1.0 / N) - mu * mu
    scale = g_ref[...] * lax.rsqrt(var + 1e-5)
    hr = jnp.maximum((h_ref[...] - mu) * scale + be_ref[...], 0.0)
    y2_ref[...] = jnp.dot(hr, wn2_ref[...], preferred_element_type=jnp.float32)
    s2b_ref[...] = (
        jnp.dot(hr, ws2_ref[...], preferred_element_type=jnp.float32)
        + b2_ref[...]
    )


def _e_body(s2b_ref, acc_ref, dinv_ref, out_ref):
    a = acc_ref[0] + acc_ref[1]
    out_ref[...] = s2b_ref[...] + a * dinv_ref[...]


def _row_spec(w):
    return pl.BlockSpec((RB, w), lambda i: (i, 0))


def _acc_spec(w):
    return pl.BlockSpec((NC, RB, w), lambda i: (0, i, 0))


def _full_spec(shape):
    nd = len(shape)
    return pl.BlockSpec(shape, lambda i: (0,) * nd)


def kernel(x, edge_index, W_self1, W_neigh1, b1, gamma1, beta1,
           W_self2, W_neigh2, b2):
    src3 = jnp.concatenate(
        [edge_index[0], jnp.zeros((EPAD,), jnp.int32)]).reshape(NW, NCH, K)
    dst3 = jnp.concatenate(
        [edge_index[1], jnp.full((EPAD,), N, jnp.int32)]).reshape(NW, NCH, K)
    z80 = jnp.zeros((ND, WA), jnp.float32)
    z64 = jnp.zeros((ND, C), jnp.float32)

    ta, tb, xs = pl.pallas_call(
        _prep_body,
        grid=(NB,),
        in_specs=[_row_spec(F_IN), _full_spec((F_IN, H)),
                  _full_spec((F_IN, H)), _full_spec((H,))],
        out_specs=[_row_spec(WA), _row_spec(HH), _row_spec(H)],
        out_shape=[jax.ShapeDtypeStruct((N, WA), jnp.float32),
                   jax.ShapeDtypeStruct((N, HH), jnp.float32),
                   jax.ShapeDtypeStruct((N, H), jnp.float32)],
    )(x, W_neigh1, W_self1, b1)

    acca = _seg_sum_80(ta, src3, dst3, z80)
    accb = _seg_sum_64(tb, src3, dst3, z64)

    h, dinv, sums, ssq = pl.pallas_call(
        _c1_body,
        grid=(NB,),
        in_specs=[_row_spec(H), _acc_spec(WA), _acc_spec(HH)],
        out_specs=[_row_spec(H), _row_spec(C),
                   _full_spec((8, H)), _full_spec((8, H))],
        out_shape=[jax.ShapeDtypeStruct((N, H), jnp.float32),
                   jax.ShapeDtypeStruct((N, C), jnp.float32),
                   jax.ShapeDtypeStruct((8, H), jnp.float32),
                   jax.ShapeDtypeStruct((8, H), jnp.float32)],
    )(xs, acca, accb)

    y2, s2b = pl.pallas_call(
        _c2_body,
        grid=(NB,),
        in_specs=[_row_spec(H), _full_spec((8, H)), _full_spec((8, H)),
                  _full_spec((H,)), _full_spec((H,)),
                  _full_spec((H, C)), _full_spec((H, C)), _full_spec((C,))],
        out_specs=[_row_spec(C), _row_spec(C)],
        out_shape=[jax.ShapeDtypeStruct((N, C), jnp.float32),
                   jax.ShapeDtypeStruct((N, C), jnp.float32)],
    )(h, sums, ssq, gamma1, beta1, W_self2, W_neigh2, b2)

    acc2 = _seg_sum_64(y2, src3, dst3, z64)

    out = pl.pallas_call(
        _e_body,
        grid=(NB,),
        in_specs=[_row_spec(C), _acc_spec(C), _row_spec(C)],
        out_specs=_row_spec(C),
        out_shape=jax.ShapeDtypeStruct((N, C), jnp.float32),
    )(s2b, acc2, dinv)

    return out


# speedup vs baseline: 2.3486x; 2.3486x over previous
import functools

import jax
import jax.numpy as jnp
from jax import lax
from jax.experimental import pallas as pl
from jax.experimental.pallas import tpu as pltpu
from jax.experimental.pallas import tpu_sc as plsc

N = 10000
E = 320000
F_IN = 128
H = 128
C = 64
HH = 64
WA = 80

NB = 10
RB = N // NB

NC = 2
NS = 16
NW = NC * NS
K = 100
NCH = 100
EPW = NCH * K
ND = N
RPT = N // NS


def _ring_pass(table, zeros, out, acc, src_v, dst_v, buf0, buf1, sem0, sem1,
               c, s):
    rz = ND // NS
    pltpu.sync_copy(zeros.at[pl.ds(s * rz, rz)], acc.at[pl.ds(s * rz, rz)])
    plsc.subcore_barrier()

    pltpu.async_copy(table.at[src_v.at[0]], buf0, sem0)
    pltpu.async_copy(table.at[src_v.at[1]], buf1, sem1)

    def body(i, carry):
        g0 = 2 * i
        g1 = 2 * i + 1
        pltpu.make_async_copy(table.at[src_v.at[g0]], buf0, sem0).wait()
        pltpu.sync_copy(buf0, acc.at[dst_v.at[g0]], add=True)

        @pl.when(g0 + 2 < NCH)
        def _():
            pltpu.async_copy(table.at[src_v.at[g0 + 2]], buf0, sem0)

        pltpu.make_async_copy(table.at[src_v.at[g1]], buf1, sem1).wait()
        pltpu.sync_copy(buf1, acc.at[dst_v.at[g1]], add=True)

        @pl.when(g1 + 2 < NCH)
        def _():
            pltpu.async_copy(table.at[src_v.at[g1 + 2]], buf1, sem1)

        return carry

    lax.fori_loop(0, NCH // 2, body, 0)

    plsc.subcore_barrier()
    pltpu.sync_copy(acc.at[pl.ds(s * RPT, RPT)],
                    out.at[c, pl.ds(s * RPT, RPT)])


def _make_seg_sum(width):
    mesh = plsc.VectorSubcoreMesh(core_axis_name="c", subcore_axis_name="s")

    @functools.partial(
        pl.kernel,
        mesh=mesh,
        out_type=jax.ShapeDtypeStruct((NC, N, width), jnp.float32),
        compiler_params=pltpu.CompilerParams(use_tc_tiling_on_sc=False),
        scratch_types=[
            pltpu.VMEM((NCH, K), jnp.int32),
            pltpu.VMEM((NCH, K), jnp.int32),
            pltpu.VMEM((K, width), jnp.float32),
            pltpu.VMEM((K, width), jnp.float32),
            pltpu.VMEM_SHARED((ND, width), jnp.float32),
            pltpu.SemaphoreType.DMA,
            pltpu.SemaphoreType.DMA,
        ],
    )
    def seg(table, src3, dst3, zeros, out, src_v, dst_v, buf0, buf1,
            acc, sem0, sem1):
        c = lax.axis_index("c")
        s = lax.axis_index("s")
        wid = c * NS + s
        pltpu.sync_copy(src3.at[wid], src_v)
        pltpu.sync_copy(dst3.at[wid], dst_v)
        _ring_pass(table, zeros, out, acc, src_v, dst_v, buf0, buf1,
                   sem0, sem1, c, s)

    return seg


_seg_sum_80 = _make_seg_sum(WA)
_seg_sum_64 = _make_seg_sum(C)


def _prep_body(x_ref, wn_ref, ws_ref, b_ref, ta_ref, tb_ref, xs_ref):
    x = x_ref[...]
    y = jnp.dot(x, wn_ref[...], preferred_element_type=jnp.float32)
    ta_ref[:, :HH] = y[:, :HH]
    lane = lax.broadcasted_iota(jnp.int32, (RB, WA - HH), 1)
    ta_ref[:, HH:] = jnp.where(lane == 0, 1.0, 0.0)
    tb_ref[...] = y[:, HH:]
    xs_ref[...] = (
        jnp.dot(x, ws_ref[...], preferred_element_type=jnp.float32) + b_ref[...]
    )


def _c1_body(xs_ref, acca_ref, accb_ref, h_ref, dinv_ref, sum_ref, ssq_ref):
    i = pl.program_id(0)
    a = acca_ref[0] + acca_ref[1]
    b = accb_ref[0] + accb_ref[1]
    deg = a[:, HH:HH + 1]
    dinv = 1.0 / jnp.maximum(deg, 1.0)
    agg = jnp.concatenate([a[:, :HH], b], axis=1)
    h = xs_ref[...] + agg * dinv
    h_ref[...] = h
    dinv_ref[...] = jnp.broadcast_to(dinv, (RB, C))

    @pl.when(i == 0)
    def _():
        sum_ref[...] = jnp.zeros_like(sum_ref)
        ssq_ref[...] = jnp.zeros_like(ssq_ref)

    sum_ref[0:1, :] += jnp.sum(h, axis=0, keepdims=True)
    ssq_ref[0:1, :] += jnp.sum(h * h, axis=0, keepdims=True)


def _c2_body(h_ref, sum_ref, ssq_ref, g_ref, be_ref, ws2_ref, wn2_ref, b2_ref,
             y2_ref, s2b_ref):
    mu = sum_ref[0, :] * (1.0 / N)
    var = ssq_ref[0, :] * (1.0 / N) - mu * mu
    scale = g_ref[...] * lax.rsqrt(var + 1e-5)
    hr = jnp.maximum((h_ref[...] - mu) * scale + be_ref[...], 0.0)
    y2_ref[...] = jnp.dot(hr, wn2_ref[...], preferred_element_type=jnp.float32)
    s2b_ref[...] = (
        jnp.dot(hr, ws2_ref[...], preferred_element_type=jnp.float32)
        + b2_ref[...]
    )


def _e_body(s2b_ref, acc_ref, dinv_ref, out_ref):
    a = acc_ref[0] + acc_ref[1]
    out_ref[...] = s2b_ref[...] + a * dinv_ref[...]


def _row_spec(w):
    return pl.BlockSpec((RB, w), lambda i: (i, 0))


def _acc_spec(w):
    return pl.BlockSpec((NC, RB, w), lambda i: (0, i, 0))


def _full_spec(shape):
    nd = len(shape)
    return pl.BlockSpec(shape, lambda i: (0,) * nd)


def kernel(x, edge_index, W_self1, W_neigh1, b1, gamma1, beta1,
           W_self2, W_neigh2, b2):
    src3 = edge_index[0].reshape(NW, NCH, K)
    dst3 = edge_index[1].reshape(NW, NCH, K)
    z80 = jnp.zeros((ND, WA), jnp.float32)
    z64 = jnp.zeros((ND, C), jnp.float32)

    ta, tb, xs = pl.pallas_call(
        _prep_body,
        grid=(NB,),
        in_specs=[_row_spec(F_IN), _full_spec((F_IN, H)),
                  _full_spec((F_IN, H)), _full_spec((H,))],
        out_specs=[_row_spec(WA), _row_spec(HH), _row_spec(H)],
        out_shape=[jax.ShapeDtypeStruct((N, WA), jnp.float32),
                   jax.ShapeDtypeStruct((N, HH), jnp.float32),
                   jax.ShapeDtypeStruct((N, H), jnp.float32)],
    )(x, W_neigh1, W_self1, b1)

    acca = _seg_sum_80(ta, src3, dst3, z80)
    accb = _seg_sum_64(tb, src3, dst3, z64)

    h, dinv, sums, ssq = pl.pallas_call(
        _c1_body,
        grid=(NB,),
        in_specs=[_row_spec(H), _acc_spec(WA), _acc_spec(HH)],
        out_specs=[_row_spec(H), _row_spec(C),
                   _full_spec((8, H)), _full_spec((8, H))],
        out_shape=[jax.ShapeDtypeStruct((N, H), jnp.float32),
                   jax.ShapeDtypeStruct((N, C), jnp.float32),
                   jax.ShapeDtypeStruct((8, H), jnp.float32),
                   jax.ShapeDtypeStruct((8, H), jnp.float32)],
    )(xs, acca, accb)

    y2, s2b = pl.pallas_call(
        _c2_body,
        grid=(NB,),
        in_specs=[_row_spec(H), _full_spec((8, H)), _full_spec((8, H)),
                  _full_spec((H,)), _full_spec((H,)),
                  _full_spec((H, C)), _full_spec((H, C)), _full_spec((C,))],
        out_specs=[_row_spec(C), _row_spec(C)],
        out_shape=[jax.ShapeDtypeStruct((N, C), jnp.float32),
                   jax.ShapeDtypeStruct((N, C), jnp.float32)],
    )(h, sums, ssq, gamma1, beta1, W_self2, W_neigh2, b2)

    acc2 = _seg_sum_64(y2, src3, dst3, z64)

    out = pl.pallas_call(
        _e_body,
        grid=(NB,),
        in_specs=[_row_spec(C), _acc_spec(C), _row_spec(C)],
        out_specs=_row_spec(C),
        out_shape=jax.ShapeDtypeStruct((N, C), jnp.float32),
    )(s2b, acc2, dinv)

    return out


# speedup vs baseline: 2.3591x; 1.0045x over previous
import functools

import jax
import jax.numpy as jnp
from jax import lax
from jax.experimental import pallas as pl
from jax.experimental.pallas import tpu as pltpu
from jax.experimental.pallas import tpu_sc as plsc

N = 10000
E = 320000
F_IN = 128
H = 128
C = 64
HH = 64
WA = 72

NB = 10
RB = N // NB

NC = 2
NS = 16
NW = NC * NS
K = 100
NCH = 100
EPW = NCH * K
ND = N
RPT = N // NS


def _ring_pass(table, zeros, out, acc, src_v, dst_v, buf0, buf1, sem0, sem1,
               c, s):
    rz = ND // NS
    pltpu.sync_copy(zeros.at[pl.ds(s * rz, rz)], acc.at[pl.ds(s * rz, rz)])
    plsc.subcore_barrier()

    pltpu.async_copy(table.at[src_v.at[0]], buf0, sem0)
    pltpu.async_copy(table.at[src_v.at[1]], buf1, sem1)

    def body(i, carry):
        g0 = 2 * i
        g1 = 2 * i + 1
        pltpu.make_async_copy(table.at[src_v.at[g0]], buf0, sem0).wait()
        pltpu.sync_copy(buf0, acc.at[dst_v.at[g0]], add=True)

        @pl.when(g0 + 2 < NCH)
        def _():
            pltpu.async_copy(table.at[src_v.at[g0 + 2]], buf0, sem0)

        pltpu.make_async_copy(table.at[src_v.at[g1]], buf1, sem1).wait()
        pltpu.sync_copy(buf1, acc.at[dst_v.at[g1]], add=True)

        @pl.when(g1 + 2 < NCH)
        def _():
            pltpu.async_copy(table.at[src_v.at[g1 + 2]], buf1, sem1)

        return carry

    lax.fori_loop(0, NCH // 2, body, 0)

    plsc.subcore_barrier()
    pltpu.sync_copy(acc.at[pl.ds(s * RPT, RPT)],
                    out.at[c, pl.ds(s * RPT, RPT)])


def _make_seg_sum(width):
    mesh = plsc.VectorSubcoreMesh(core_axis_name="c", subcore_axis_name="s")

    @functools.partial(
        pl.kernel,
        mesh=mesh,
        out_type=jax.ShapeDtypeStruct((NC, N, width), jnp.float32),
        compiler_params=pltpu.CompilerParams(use_tc_tiling_on_sc=False),
        scratch_types=[
            pltpu.VMEM((NCH, K), jnp.int32),
            pltpu.VMEM((NCH, K), jnp.int32),
            pltpu.VMEM((K, width), jnp.float32),
            pltpu.VMEM((K, width), jnp.float32),
            pltpu.VMEM_SHARED((ND, width), jnp.float32),
            pltpu.SemaphoreType.DMA,
            pltpu.SemaphoreType.DMA,
        ],
    )
    def seg(table, src3, dst3, zeros, out, src_v, dst_v, buf0, buf1,
            acc, sem0, sem1):
        c = lax.axis_index("c")
        s = lax.axis_index("s")
        wid = c * NS + s
        pltpu.sync_copy(src3.at[wid], src_v)
        pltpu.sync_copy(dst3.at[wid], dst_v)
        _ring_pass(table, zeros, out, acc, src_v, dst_v, buf0, buf1,
                   sem0, sem1, c, s)

    return seg


_seg_sum_80 = _make_seg_sum(WA)
_seg_sum_64 = _make_seg_sum(C)


def _prep_body(x_ref, wn_ref, ws_ref, b_ref, ta_ref, tb_ref, xs_ref):
    x = x_ref[...]
    y = jnp.dot(x, wn_ref[...], preferred_element_type=jnp.float32)
    ta_ref[:, :HH] = y[:, :HH]
    lane = lax.broadcasted_iota(jnp.int32, (RB, WA - HH), 1)
    ta_ref[:, HH:] = jnp.where(lane == 0, 1.0, 0.0)
    tb_ref[...] = y[:, HH:]
    xs_ref[...] = (
        jnp.dot(x, ws_ref[...], preferred_element_type=jnp.float32) + b_ref[...]
    )


def _c1_body(xs_ref, acca_ref, accb_ref, h_ref, dinv_ref, sum_ref, ssq_ref):
    i = pl.program_id(0)
    a = acca_ref[0] + acca_ref[1]
    b = accb_ref[0] + accb_ref[1]
    deg = a[:, HH:HH + 1]
    dinv = 1.0 / jnp.maximum(deg, 1.0)
    agg = jnp.concatenate([a[:, :HH], b], axis=1)
    h = xs_ref[...] + agg * dinv
    h_ref[...] = h
    dinv_ref[...] = jnp.broadcast_to(dinv, (RB, C))

    @pl.when(i == 0)
    def _():
        sum_ref[...] = jnp.zeros_like(sum_ref)
        ssq_ref[...] = jnp.zeros_like(ssq_ref)

    sum_ref[0:1, :] += jnp.sum(h, axis=0, keepdims=True)
    ssq_ref[0:1, :] += jnp.sum(h * h, axis=0, keepdims=True)


def _c2_body(h_ref, sum_ref, ssq_ref, g_ref, be_ref, ws2_ref, wn2_ref, b2_ref,
             y2_ref, s2b_ref):
    mu = sum_ref[0, :] * (1.0 / N)
    var = ssq_ref[0, :] * (1.0 / N) - mu * mu
    scale = g_ref[...] * lax.rsqrt(var + 1e-5)
    hr = jnp.maximum((h_ref[...] - mu) * scale + be_ref[...], 0.0)
    y2_ref[...] = jnp.dot(hr, wn2_ref[...], preferred_element_type=jnp.float32)
    s2b_ref[...] = (
        jnp.dot(hr, ws2_ref[...], preferred_element_type=jnp.float32)
        + b2_ref[...]
    )


def _e_body(s2b_ref, acc_ref, dinv_ref, out_ref):
    a = acc_ref[0] + acc_ref[1]
    out_ref[...] = s2b_ref[...] + a * dinv_ref[...]


def _row_spec(w):
    return pl.BlockSpec((RB, w), lambda i: (i, 0))


def _acc_spec(w):
    return pl.BlockSpec((NC, RB, w), lambda i: (0, i, 0))


def _full_spec(shape):
    nd = len(shape)
    return pl.BlockSpec(shape, lambda i: (0,) * nd)


def kernel(x, edge_index, W_self1, W_neigh1, b1, gamma1, beta1,
           W_self2, W_neigh2, b2):
    src3 = edge_index[0].reshape(NW, NCH, K)
    dst3 = edge_index[1].reshape(NW, NCH, K)
    z80 = jnp.zeros((ND, WA), jnp.float32)
    z64 = jnp.zeros((ND, C), jnp.float32)

    ta, tb, xs = pl.pallas_call(
        _prep_body,
        grid=(NB,),
        in_specs=[_row_spec(F_IN), _full_spec((F_IN, H)),
                  _full_spec((F_IN, H)), _full_spec((H,))],
        out_specs=[_row_spec(WA), _row_spec(HH), _row_spec(H)],
        out_shape=[jax.ShapeDtypeStruct((N, WA), jnp.float32),
                   jax.ShapeDtypeStruct((N, HH), jnp.float32),
                   jax.ShapeDtypeStruct((N, H), jnp.float32)],
    )(x, W_neigh1, W_self1, b1)

    acca = _seg_sum_80(ta, src3, dst3, z80)
    accb = _seg_sum_64(tb, src3, dst3, z64)

    h, dinv, sums, ssq = pl.pallas_call(
        _c1_body,
        grid=(NB,),
        in_specs=[_row_spec(H), _acc_spec(WA), _acc_spec(HH)],
        out_specs=[_row_spec(H), _row_spec(C),
                   _full_spec((8, H)), _full_spec((8, H))],
        out_shape=[jax.ShapeDtypeStruct((N, H), jnp.float32),
                   jax.ShapeDtypeStruct((N, C), jnp.float32),
                   jax.ShapeDtypeStruct((8, H), jnp.float32),
                   jax.ShapeDtypeStruct((8, H), jnp.float32)],
    )(xs, acca, accb)

    y2, s2b = pl.pallas_call(
        _c2_body,
        grid=(NB,),
        in_specs=[_row_spec(H), _full_spec((8, H)), _full_spec((8, H)),
                  _full_spec((H,)), _full_spec((H,)),
                  _full_spec((H, C)), _full_spec((H, C)), _full_spec((C,))],
        out_specs=[_row_spec(C), _row_spec(C)],
        out_shape=[jax.ShapeDtypeStruct((N, C), jnp.float32),
                   jax.ShapeDtypeStruct((N, C), jnp.float32)],
    )(h, sums, ssq, gamma1, beta1, W_self2, W_neigh2, b2)

    acc2 = _seg_sum_64(y2, src3, dst3, z64)

    out = pl.pallas_call(
        _e_body,
        grid=(NB,),
        in_specs=[_row_spec(C), _acc_spec(C), _row_spec(C)],
        out_specs=_row_spec(C),
        out_shape=jax.ShapeDtypeStruct((N, C), jnp.float32),
    )(s2b, acc2, dinv)

    return out


# speedup vs baseline: 2.7139x; 1.1504x over previous
import functools

import jax
import jax.numpy as jnp
from jax import lax
from jax.experimental import pallas as pl
from jax.experimental.pallas import tpu as pltpu
from jax.experimental.pallas import tpu_sc as plsc

N = 10000
E = 320000
F_IN = 128
H = 128
C = 64
WA = 144

NB = 10
RB = N // NB

NC = 2
NS = 16
NW = NC * NS
K = 100
NCH = 100
EPW = NCH * K
ND = N
RPT = N // NS


def _ring_pass(table, zeros, out, acc, src_v, dst_v, buf0, buf1, sem0, sem1,
               c, s):
    rz = ND // NS
    pltpu.sync_copy(zeros.at[pl.ds(s * rz, rz)], acc.at[pl.ds(s * rz, rz)])
    plsc.subcore_barrier()

    pltpu.async_copy(table.at[src_v.at[0]], buf0, sem0)
    pltpu.async_copy(table.at[src_v.at[1]], buf1, sem1)

    def body(i, carry):
        g0 = 2 * i
        g1 = 2 * i + 1
        pltpu.make_async_copy(table.at[src_v.at[g0]], buf0, sem0).wait()
        pltpu.sync_copy(buf0, acc.at[dst_v.at[g0]], add=True)

        @pl.when(g0 + 2 < NCH)
        def _():
            pltpu.async_copy(table.at[src_v.at[g0 + 2]], buf0, sem0)

        pltpu.make_async_copy(table.at[src_v.at[g1]], buf1, sem1).wait()
        pltpu.sync_copy(buf1, acc.at[dst_v.at[g1]], add=True)

        @pl.when(g1 + 2 < NCH)
        def _():
            pltpu.async_copy(table.at[src_v.at[g1 + 2]], buf1, sem1)

        return carry

    lax.fori_loop(0, NCH // 2, body, 0)

    plsc.subcore_barrier()
    pltpu.sync_copy(acc.at[pl.ds(s * RPT, RPT)],
                    out.at[c, pl.ds(s * RPT, RPT)])


def _make_seg_sum(width):
    mesh = plsc.VectorSubcoreMesh(core_axis_name="c", subcore_axis_name="s")

    @functools.partial(
        pl.kernel,
        mesh=mesh,
        out_type=jax.ShapeDtypeStruct((NC, N, width), jnp.bfloat16),
        compiler_params=pltpu.CompilerParams(use_tc_tiling_on_sc=False),
        scratch_types=[
            pltpu.VMEM((NCH, K), jnp.int32),
            pltpu.VMEM((NCH, K), jnp.int32),
            pltpu.VMEM((K, width), jnp.bfloat16),
            pltpu.VMEM((K, width), jnp.bfloat16),
            pltpu.VMEM_SHARED((ND, width), jnp.bfloat16),
            pltpu.SemaphoreType.DMA,
            pltpu.SemaphoreType.DMA,
        ],
    )
    def seg(table, src3, dst3, zeros, out, src_v, dst_v, buf0, buf1,
            acc, sem0, sem1):
        c = lax.axis_index("c")
        s = lax.axis_index("s")
        wid = c * NS + s
        pltpu.sync_copy(src3.at[wid], src_v)
        pltpu.sync_copy(dst3.at[wid], dst_v)
        _ring_pass(table, zeros, out, acc, src_v, dst_v, buf0, buf1,
                   sem0, sem1, c, s)

    return seg


_seg_sum_144 = _make_seg_sum(WA)
_seg_sum_64 = _make_seg_sum(C)


def _prep_body(x_ref, wn_ref, ws_ref, b_ref, ta_ref, xs_ref):
    x = x_ref[...]
    y = jnp.dot(x, wn_ref[...], preferred_element_type=jnp.float32)
    ta_ref[:, :H] = y.astype(jnp.bfloat16)
    lane = lax.broadcasted_iota(jnp.int32, (RB, WA - H), 1)
    ta_ref[:, H:] = jnp.where(lane == 0, 1.0, 0.0).astype(jnp.bfloat16)
    xs_ref[...] = (
        jnp.dot(x, ws_ref[...], preferred_element_type=jnp.float32) + b_ref[...]
    )


def _c1_body(xs_ref, acc_ref, h_ref, dinv_ref, sum_ref, ssq_ref):
    i = pl.program_id(0)
    a = (acc_ref[0].astype(jnp.float32) + acc_ref[1].astype(jnp.float32))
    deg = a[:, H:H + 1]
    dinv = 1.0 / jnp.maximum(deg, 1.0)
    h = xs_ref[...] + a[:, :H] * dinv
    h_ref[...] = h
    dinv_ref[...] = jnp.broadcast_to(dinv, (RB, C))

    @pl.when(i == 0)
    def _():
        sum_ref[...] = jnp.zeros_like(sum_ref)
        ssq_ref[...] = jnp.zeros_like(ssq_ref)

    sum_ref[0:1, :] += jnp.sum(h, axis=0, keepdims=True)
    ssq_ref[0:1, :] += jnp.sum(h * h, axis=0, keepdims=True)


def _c2_body(h_ref, sum_ref, ssq_ref, g_ref, be_ref, ws2_ref, wn2_ref, b2_ref,
             y2_ref, s2b_ref):
    mu = sum_ref[0, :] * (1.0 / N)
    var = ssq_ref[0, :] * (1.0 / N) - mu * mu
    scale = g_ref[...] * lax.rsqrt(var + 1e-5)
    hr = jnp.maximum((h_ref[...] - mu) * scale + be_ref[...], 0.0)
    y2 = jnp.dot(hr, wn2_ref[...], preferred_element_type=jnp.float32)
    y2_ref[...] = y2.astype(jnp.bfloat16)
    s2b_ref[...] = (
        jnp.dot(hr, ws2_ref[...], preferred_element_type=jnp.float32)
        + b2_ref[...]
    )


def _e_body(s2b_ref, acc_ref, dinv_ref, out_ref):
    a = acc_ref[0].astype(jnp.float32) + acc_ref[1].astype(jnp.float32)
    out_ref[...] = s2b_ref[...] + a * dinv_ref[...]


def _row_spec(w):
    return pl.BlockSpec((RB, w), lambda i: (i, 0))


def _acc_spec(w):
    return pl.BlockSpec((NC, RB, w), lambda i: (0, i, 0))


def _full_spec(shape):
    nd = len(shape)
    return pl.BlockSpec(shape, lambda i: (0,) * nd)


def kernel(x, edge_index, W_self1, W_neigh1, b1, gamma1, beta1,
           W_self2, W_neigh2, b2):
    src3 = edge_index[0].reshape(NW, NCH, K)
    dst3 = edge_index[1].reshape(NW, NCH, K)
    z144 = jnp.zeros((ND, WA), jnp.bfloat16)
    z64 = jnp.zeros((ND, C), jnp.bfloat16)

    ta, xs = pl.pallas_call(
        _prep_body,
        grid=(NB,),
        in_specs=[_row_spec(F_IN), _full_spec((F_IN, H)),
                  _full_spec((F_IN, H)), _full_spec((H,))],
        out_specs=[_row_spec(WA), _row_spec(H)],
        out_shape=[jax.ShapeDtypeStruct((N, WA), jnp.bfloat16),
                   jax.ShapeDtypeStruct((N, H), jnp.float32)],
    )(x, W_neigh1, W_self1, b1)

    acc1 = _seg_sum_144(ta, src3, dst3, z144)

    h, dinv, sums, ssq = pl.pallas_call(
        _c1_body,
        grid=(NB,),
        in_specs=[_row_spec(H), _acc_spec(WA)],
        out_specs=[_row_spec(H), _row_spec(C),
                   _full_spec((8, H)), _full_spec((8, H))],
        out_shape=[jax.ShapeDtypeStruct((N, H), jnp.float32),
                   jax.ShapeDtypeStruct((N, C), jnp.float32),
                   jax.ShapeDtypeStruct((8, H), jnp.float32),
                   jax.ShapeDtypeStruct((8, H), jnp.float32)],
    )(xs, acc1)

    y2, s2b = pl.pallas_call(
        _c2_body,
        grid=(NB,),
        in_specs=[_row_spec(H), _full_spec((8, H)), _full_spec((8, H)),
                  _full_spec((H,)), _full_spec((H,)),
                  _full_spec((H, C)), _full_spec((H, C)), _full_spec((C,))],
        out_specs=[_row_spec(C), _row_spec(C)],
        out_shape=[jax.ShapeDtypeStruct((N, C), jnp.bfloat16),
                   jax.ShapeDtypeStruct((N, C), jnp.float32)],
    )(h, sums, ssq, gamma1, beta1, W_self2, W_neigh2, b2)

    acc2 = _seg_sum_64(y2, src3, dst3, z64)

    out = pl.pallas_call(
        _e_body,
        grid=(NB,),
        in_specs=[_row_spec(C), _acc_spec(C), _row_spec(C)],
        out_specs=_row_spec(C),
        out_shape=jax.ShapeDtypeStruct((N, C), jnp.float32),
    )(s2b, acc2, dinv)

    return out
